# baseline (speedup 1.0000x reference)
import functools
import math

import jax
import jax.numpy as jnp
from jax import lax
from jax.experimental import pallas as pl
from jax.experimental.pallas import tpu as pltpu
from jax.experimental.pallas import tpu_sc as plsc

_VOCAB = 1000000
_EMB = 64
_B = 4096
_L = 200

_NC = 2
_NS = 16
_NW = _NC * _NS
_CB = _B // _NW
_NK = _CB // 16
_ROWS = _VOCAB // 2
_SCALE = math.sqrt(float(_EMB))


_NJ = (_VOCAB + 127) // 128
_ROWS2 = 64 * _NJ


def _make_detile_kernel():
    mesh = plsc.VectorSubcoreMesh(core_axis_name="c", subcore_axis_name="s")

    @functools.partial(
        pl.kernel,
        mesh=mesh,
        out_type=jax.ShapeDtypeStruct((_ROWS2, 128), jnp.float32),
        scratch_types=[
            pltpu.VMEM((_EMB, 128), jnp.float32),
            pltpu.VMEM((_EMB, 128), jnp.float32),
            pltpu.VMEM((_EMB, 128), jnp.float32),
            pltpu.VMEM((_EMB, 128), jnp.float32),
            pltpu.SemaphoreType.DMA,
            pltpu.SemaphoreType.DMA,
            pltpu.SemaphoreType.DMA,
            pltpu.SemaphoreType.DMA,
        ],
        compiler_params=pltpu.CompilerParams(needs_layout_passes=False),
    )
    def sc_detile(tt_hbm, lt_hbm, gbuf0, gbuf1, obuf0, obuf1,
                  r0, r1, w0, w1):
        wid = lax.axis_index("s") * _NC + lax.axis_index("c")
        base = 244 * wid + jnp.minimum(wid, _NJ - 244 * _NW)
        nw = 244 + (wid < _NJ - 244 * _NW).astype(jnp.int32)

        gbufs = (gbuf0, gbuf1)
        obufs = (obuf0, obuf1)
        rsems = (r0, r1)
        wsems = (w0, w1)
        lanes = lax.iota(jnp.int32, 16)
        rows4 = tuple(lanes + 16 * q for q in range(4))

        def offs(s):
            j = base + s
            return (pl.multiple_of(128 * j, 128),
                    pl.multiple_of(64 * j, 64))

        def start_read(s, b):
            in_off, _ = offs(s)
            pltpu.async_copy(
                tt_hbm.at[:, pl.ds(in_off, 128)], gbufs[b], rsems[b])

        def wait_read(s, b):
            in_off, _ = offs(s)
            pltpu.make_async_copy(
                tt_hbm.at[:, pl.ds(in_off, 128)], gbufs[b], rsems[b]).wait()

        def transpose(b):
            g = gbufs[b]
            o = obufs[b]

            def rg_body(rg, _):
                c0 = 32 * rg
                for rr in range(16):
                    vs = []
                    for m in range(8):
                        col = jnp.full((16,), c0 + 2 * rr + m // 4,
                                       jnp.int32)
                        vs.append(
                            plsc.load_gather(g, [rows4[m % 4], col]))
                    for m in range(8):
                        o[16 * rg + rr, pl.ds(16 * m, 16)] = vs[m]
                return 0

            lax.fori_loop(0, 4, rg_body, 0)

        def start_write(s, b):
            _, out_off = offs(s)
            pltpu.async_copy(
                obufs[b], lt_hbm.at[pl.ds(out_off, _EMB), :], wsems[b])

        def wait_write(s, b):
            _, out_off = offs(s)
            pltpu.make_async_copy(
                obufs[b], lt_hbm.at[pl.ds(out_off, _EMB), :],
                wsems[b]).wait()

        start_read(0, 0)
        start_read(1, 1)

        def body(i, _):
            for b in range(2):
                s = 2 * i + b

                @pl.when(s < nw)
                def _():
                    wait_read(s, b)

                @pl.when(jnp.logical_and(s < nw, s >= 2))
                def _():
                    wait_write(s - 2, b)

                @pl.when(s < nw)
                def _():
                    transpose(b)

                @pl.when(s + 2 < nw)
                def _():
                    start_read(s + 2, b)

                @pl.when(s < nw)
                def _():
                    start_write(s, b)
            return 0

        lax.fori_loop(0, 123, body, 0)

        for b in range(2):
            wait_write(nw - 2 + b, b)

    return sc_detile


def _make_sc_kernel():
    mesh = plsc.VectorSubcoreMesh(core_axis_name="c", subcore_axis_name="s")

    @functools.partial(
        pl.kernel,
        mesh=mesh,
        out_type=jax.ShapeDtypeStruct((_L, _EMB, _B), jnp.float32),
        scratch_types=[
            pltpu.VMEM((_L, _CB), jnp.int32),
            pltpu.VMEM((_CB, 128), jnp.float32),
            pltpu.VMEM((_CB, 128), jnp.float32),
            pltpu.VMEM((_EMB, _CB), jnp.float32),
            pltpu.VMEM((_EMB, _CB), jnp.float32),
            pltpu.VMEM((_CB,), jnp.int32),
            pltpu.VMEM((_CB,), jnp.int32),
            pltpu.SemaphoreType.DMA,
            pltpu.SemaphoreType.DMA,
            pltpu.SemaphoreType.DMA,
            pltpu.SemaphoreType.DMA,
        ],
        compiler_params=pltpu.CompilerParams(needs_layout_passes=False),
    )
    def sc_embed(tok_hbm, lt_hbm, out_hbm,
                 tok_v, gbuf0, gbuf1, obuf0, obuf1, idx0, idx1,
                 g0, g1, w0, w1):
        wid = lax.axis_index("s") * _NC + lax.axis_index("c")
        band = wid * _CB

        pltpu.sync_copy(tok_hbm.at[:, pl.ds(band, _CB)], tok_v)

        gbufs = (gbuf0, gbuf1)
        obufs = (obuf0, obuf1)
        idxs = (idx0, idx1)
        gsems = (g0, g1)
        wsems = (w0, w1)
        lanes = lax.iota(jnp.int32, 16)
        rows = tuple(lanes + 16 * k for k in range(_NK))

        def fill_idx(l, b):
            for k in range(_NK):
                sl = pl.ds(16 * k, 16)
                idxs[b][sl] = lax.shift_right_logical(tok_v[l, sl], 1)

        def start_gather(b):
            pltpu.async_copy(lt_hbm.at[idxs[b]], gbufs[b], gsems[b])

        def wait_gather(b):
            pltpu.make_async_copy(
                lt_hbm.at[idxs[b]], gbufs[b], gsems[b]).wait()

        def transpose_scale(l, b):
            g = gbufs[b]
            o = obufs[b]

            def eg_body(eg, _):
                e0 = eg * 16
                for k in range(_NK):
                    sl = pl.ds(16 * k, 16)
                    he = lax.bitwise_or(
                        lax.shift_left(
                            lax.bitwise_and(tok_v[l, sl], 1), 6), e0)
                    vs = [
                        plsc.load_gather(g, [rows[k], lax.bitwise_or(he, ei)])
                        for ei in range(16)
                    ]
                    for ei in range(16):
                        o[e0 + ei, sl] = vs[ei] * _SCALE
                return 0

            lax.fori_loop(0, _EMB // 16, eg_body, 0)

        def start_write(l, b):
            pltpu.async_copy(
                obufs[b], out_hbm.at[l, :, pl.ds(band, _CB)], wsems[b])

        def wait_write(l, b):
            pltpu.make_async_copy(
                obufs[b], out_hbm.at[l, :, pl.ds(band, _CB)], wsems[b]).wait()

        fill_idx(0, 0)
        start_gather(0)
        fill_idx(1, 1)
        start_gather(1)

        def body(i, _):
            for b in range(2):
                l = 2 * i + b
                wait_gather(b)

                @pl.when(i >= 1)
                def _():
                    wait_write(l - 2, b)

                transpose_scale(l, b)

                @pl.when(i < _L // 2 - 1)
                def _():
                    fill_idx(l + 2, b)
                    start_gather(b)

                start_write(l, b)
            return 0

        lax.fori_loop(0, _L // 2, body, 0)

        for b in range(2):
            wait_write(_L - 2 + b, b)

    return sc_embed


_sc_detile = _make_detile_kernel()
_sc_embed = _make_sc_kernel()


def kernel(tokens, table):
    tok_t = tokens.T
    tt = table.T
    lt = _sc_detile(tt)
    out = _sc_embed(tok_t, lt)
    return out.transpose(2, 0, 1)


# speedup vs baseline: 1.4132x; 1.4132x over previous
import functools
import math

import jax
import jax.numpy as jnp
from jax import lax
from jax.experimental import pallas as pl
from jax.experimental.pallas import tpu as pltpu
from jax.experimental.pallas import tpu_sc as plsc

_VOCAB = 1000000
_EMB = 64
_B = 4096
_L = 200

_NC = 2
_NS = 16
_NW = _NC * _NS
_CB = _B // _NW
_NK = _CB // 16
_ROWS = _VOCAB // 2
_SCALE = math.sqrt(float(_EMB))


def _make_sc_kernel():
    mesh = plsc.VectorSubcoreMesh(core_axis_name="c", subcore_axis_name="s")

    @functools.partial(
        pl.kernel,
        mesh=mesh,
        out_type=jax.ShapeDtypeStruct((_L, _EMB, _B), jnp.float32),
        scratch_types=[
            pltpu.VMEM((_L, _CB), jnp.int32),
            pltpu.VMEM((_CB, 128), jnp.float32),
            pltpu.VMEM((_CB, 128), jnp.float32),
            pltpu.VMEM((_EMB, _CB), jnp.float32),
            pltpu.VMEM((_EMB, _CB), jnp.float32),
            pltpu.VMEM((_CB,), jnp.int32),
            pltpu.VMEM((_CB,), jnp.int32),
            pltpu.SemaphoreType.DMA,
            pltpu.SemaphoreType.DMA,
            pltpu.SemaphoreType.DMA,
            pltpu.SemaphoreType.DMA,
        ],
        compiler_params=pltpu.CompilerParams(needs_layout_passes=False),
    )
    def sc_embed(tok_hbm, lt_hbm, out_hbm,
                 tok_v, gbuf0, gbuf1, obuf0, obuf1, idx0, idx1,
                 g0, g1, w0, w1):
        wid = lax.axis_index("s") * _NC + lax.axis_index("c")
        band = wid * _CB

        pltpu.sync_copy(tok_hbm.at[:, pl.ds(band, _CB)], tok_v)

        gbufs = (gbuf0, gbuf1)
        obufs = (obuf0, obuf1)
        idxs = (idx0, idx1)
        gsems = (g0, g1)
        wsems = (w0, w1)
        lanes = lax.iota(jnp.int32, 16)
        rows = tuple(lanes + 16 * k for k in range(_NK))

        def fill_idx(l, b):
            for k in range(_NK):
                sl = pl.ds(16 * k, 16)
                idxs[b][sl] = lax.shift_right_logical(tok_v[l, sl], 1)

        def start_gather(b):
            pltpu.async_copy(lt_hbm.at[idxs[b]], gbufs[b], gsems[b])

        def wait_gather(b):
            pltpu.make_async_copy(
                lt_hbm.at[idxs[b]], gbufs[b], gsems[b]).wait()

        def transpose_scale(l, b):
            g = gbufs[b]
            o = obufs[b]

            def eg_body(eg, _):
                e0 = eg * 16
                for k in range(_NK):
                    sl = pl.ds(16 * k, 16)
                    he = lax.bitwise_or(
                        lax.shift_left(
                            lax.bitwise_and(tok_v[l, sl], 1), 6), e0)
                    vs = [
                        plsc.load_gather(g, [rows[k], lax.bitwise_or(he, ei)])
                        for ei in range(16)
                    ]
                    for ei in range(16):
                        o[e0 + ei, sl] = vs[ei] * _SCALE
                return 0

            lax.fori_loop(0, _EMB // 16, eg_body, 0)

        def start_write(l, b):
            pltpu.async_copy(
                obufs[b], out_hbm.at[l, :, pl.ds(band, _CB)], wsems[b])

        def wait_write(l, b):
            pltpu.make_async_copy(
                obufs[b], out_hbm.at[l, :, pl.ds(band, _CB)], wsems[b]).wait()

        fill_idx(0, 0)
        start_gather(0)
        fill_idx(1, 1)
        start_gather(1)

        def body(i, _):
            for b in range(2):
                l = 2 * i + b
                wait_gather(b)

                @pl.when(i >= 1)
                def _():
                    wait_write(l - 2, b)

                transpose_scale(l, b)

                @pl.when(i < _L // 2 - 1)
                def _():
                    fill_idx(l + 2, b)
                    start_gather(b)

                start_write(l, b)
            return 0

        lax.fori_loop(0, _L // 2, body, 0)

        for b in range(2):
            wait_write(_L - 2 + b, b)

    return sc_embed


_sc_embed = _make_sc_kernel()


def kernel(tokens, table):
    tok_t = tokens.T
    lt = table.reshape(_ROWS, 128)
    out = _sc_embed(tok_t, lt)
    return out.transpose(2, 0, 1)


# speedup vs baseline: 2.2099x; 1.5638x over previous
import functools
import math

import jax
import jax.numpy as jnp
from jax import lax
from jax.experimental import pallas as pl
from jax.experimental.pallas import tpu as pltpu
from jax.experimental.pallas import tpu_sc as plsc

_VOCAB = 1000000
_EMB = 64
_B = 4096
_L = 200

_NC = 2
_NS = 16
_NW = _NC * _NS
_CB = _B // _NW
_NK = _CB // 16
_ROWS = _VOCAB // 2
_SCALE = math.sqrt(float(_EMB))


def _make_sc_kernel():
    mesh = plsc.VectorSubcoreMesh(core_axis_name="c", subcore_axis_name="s")

    @functools.partial(
        pl.kernel,
        mesh=mesh,
        out_type=jax.ShapeDtypeStruct((_L, _EMB, _B), jnp.float32),
        scratch_types=[
            pltpu.VMEM((_L, _CB), jnp.int32),
            pltpu.VMEM((_CB, 128), jnp.float32),
            pltpu.VMEM((_CB, 128), jnp.float32),
            pltpu.VMEM((_EMB, _CB), jnp.float32),
            pltpu.VMEM((_EMB, _CB), jnp.float32),
            pltpu.VMEM((_CB,), jnp.int32),
            pltpu.VMEM((_CB,), jnp.int32),
            pltpu.SemaphoreType.DMA,
            pltpu.SemaphoreType.DMA,
            pltpu.SemaphoreType.DMA,
            pltpu.SemaphoreType.DMA,
        ],
        compiler_params=pltpu.CompilerParams(needs_layout_passes=False),
    )
    def sc_embed(tok_hbm, lt_hbm, out_hbm,
                 tok_v, gbuf0, gbuf1, obuf0, obuf1, idx0, idx1,
                 g0, g1, w0, w1):
        wid = lax.axis_index("s") * _NC + lax.axis_index("c")
        band = wid * _CB

        pltpu.sync_copy(tok_hbm.at[:, pl.ds(band, _CB)], tok_v)

        gbufs = (gbuf0, gbuf1)
        obufs = (obuf0, obuf1)
        idxs = (idx0, idx1)
        gsems = (g0, g1)
        wsems = (w0, w1)
        lanes = lax.iota(jnp.int32, 16)
        rows = tuple(lanes + 16 * k for k in range(_NK))

        def fill_idx(l, b):
            for k in range(_NK):
                sl = pl.ds(16 * k, 16)
                idxs[b][sl] = tok_v[l, sl]

        def start_gather(b):
            pltpu.async_copy(lt_hbm.at[idxs[b]], gbufs[b], gsems[b])

        def wait_gather(b):
            pltpu.make_async_copy(
                lt_hbm.at[idxs[b]], gbufs[b], gsems[b]).wait()

        diag_cols = tuple(
            lax.rem(lanes + d, jnp.int32(16)) for d in range(16))

        def transpose_scale(l, b):
            g = gbufs[b]
            o = obufs[b]

            def eg_body(eg, _):
                e0 = eg * 16
                for k in range(_NK):
                    vs = [
                        plsc.load_gather(g, [rows[k], diag_cols[d] + e0])
                        for d in range(16)
                    ]
                    for d in range(16):
                        plsc.store_scatter(
                            o, [diag_cols[d] + e0, rows[k]],
                            vs[d] * _SCALE)
                return 0

            lax.fori_loop(0, _EMB // 16, eg_body, 0)

        def start_write(l, b):
            pltpu.async_copy(
                obufs[b], out_hbm.at[l, :, pl.ds(band, _CB)], wsems[b])

        def wait_write(l, b):
            pltpu.make_async_copy(
                obufs[b], out_hbm.at[l, :, pl.ds(band, _CB)], wsems[b]).wait()

        fill_idx(0, 0)
        start_gather(0)
        fill_idx(1, 1)
        start_gather(1)

        def body(i, _):
            for b in range(2):
                l = 2 * i + b
                wait_gather(b)

                @pl.when(i >= 1)
                def _():
                    wait_write(l - 2, b)

                transpose_scale(l, b)

                @pl.when(i < _L // 2 - 1)
                def _():
                    fill_idx(l + 2, b)
                    start_gather(b)

                start_write(l, b)
            return 0

        lax.fori_loop(0, _L // 2, body, 0)

        for b in range(2):
            wait_write(_L - 2 + b, b)

    return sc_embed


_sc_embed = _make_sc_kernel()


def kernel(tokens, table):
    tok_t = tokens.T
    ltp = jnp.pad(table, ((0, 0), (0, 64)))
    out = _sc_embed(tok_t, ltp)
    return out.transpose(2, 0, 1)


# speedup vs baseline: 2.4983x; 1.1305x over previous
import functools
import math

import jax
import jax.numpy as jnp
from jax import lax
from jax.experimental import pallas as pl
from jax.experimental.pallas import tpu as pltpu
from jax.experimental.pallas import tpu_sc as plsc

_VOCAB = 1000000
_EMB = 64
_B = 4096
_L = 200

_NC = 2
_NS = 16
_NW = _NC * _NS
_CB = _B // _NW
_NK = _CB // 16
_ROWS = _VOCAB // 2
_SCALE = math.sqrt(float(_EMB))


def _make_sc_kernel():
    mesh = plsc.VectorSubcoreMesh(core_axis_name="c", subcore_axis_name="s")

    @functools.partial(
        pl.kernel,
        mesh=mesh,
        out_type=jax.ShapeDtypeStruct((_L, _EMB, _B), jnp.float32),
        scratch_types=[
            pltpu.VMEM((_L, _CB), jnp.int32),
            pltpu.VMEM((_CB, 128), jnp.float32),
            pltpu.VMEM((_CB, 128), jnp.float32),
            pltpu.VMEM((_CB, 128), jnp.float32),
            pltpu.VMEM((_CB, 128), jnp.float32),
            pltpu.VMEM((_EMB, _CB), jnp.float32),
            pltpu.VMEM((_EMB, _CB), jnp.float32),
            pltpu.VMEM((_CB,), jnp.int32),
            pltpu.VMEM((_CB,), jnp.int32),
            pltpu.VMEM((_CB,), jnp.int32),
            pltpu.VMEM((_CB,), jnp.int32),
            pltpu.SemaphoreType.DMA,
            pltpu.SemaphoreType.DMA,
            pltpu.SemaphoreType.DMA,
            pltpu.SemaphoreType.DMA,
            pltpu.SemaphoreType.DMA,
            pltpu.SemaphoreType.DMA,
        ],
        compiler_params=pltpu.CompilerParams(needs_layout_passes=False),
    )
    def sc_embed(tok_hbm, lt_hbm, out_hbm,
                 tok_v, gbuf0, gbuf1, gbuf2, gbuf3, obuf0, obuf1,
                 idx0, idx1, idx2, idx3, g0, g1, g2, g3, w0, w1):
        wid = lax.axis_index("s") * _NC + lax.axis_index("c")
        band = wid * _CB

        pltpu.sync_copy(tok_hbm.at[:, pl.ds(band, _CB)], tok_v)

        gbufs = (gbuf0, gbuf1, gbuf2, gbuf3)
        obufs = (obuf0, obuf1)
        idxs = (idx0, idx1, idx2, idx3)
        gsems = (g0, g1, g2, g3)
        wsems = (w0, w1)
        lanes = lax.iota(jnp.int32, 16)
        rows = tuple(lanes + 16 * k for k in range(_NK))

        def fill_idx(l, b):
            for k in range(_NK):
                sl = pl.ds(16 * k, 16)
                idxs[b][sl] = tok_v[l, sl]

        def start_gather(b):
            pltpu.async_copy(lt_hbm.at[idxs[b]], gbufs[b], gsems[b])

        def wait_gather(b):
            pltpu.make_async_copy(
                lt_hbm.at[idxs[b]], gbufs[b], gsems[b]).wait()

        diag_cols = tuple(
            lax.rem(lanes + d, jnp.int32(16)) for d in range(16))

        def transpose_scale(l, s, b):
            g = gbufs[s]
            o = obufs[b]

            def eg_body(eg, _):
                e0 = eg * 16
                for k in range(_NK):
                    vs = [
                        plsc.load_gather(g, [rows[k], diag_cols[d] + e0])
                        for d in range(16)
                    ]
                    for d in range(16):
                        plsc.store_scatter(
                            o, [diag_cols[d] + e0, rows[k]],
                            vs[d] * _SCALE)
                return 0

            lax.fori_loop(0, _EMB // 16, eg_body, 0)

        def start_write(l, b):
            pltpu.async_copy(
                obufs[b], out_hbm.at[l, :, pl.ds(band, _CB)], wsems[b])

        def wait_write(l, b):
            pltpu.make_async_copy(
                obufs[b], out_hbm.at[l, :, pl.ds(band, _CB)], wsems[b]).wait()

        for s in range(4):
            fill_idx(s, s)
            start_gather(s)

        def body(i, _):
            for s in range(4):
                l = 4 * i + s
                b = s % 2
                wait_gather(s)

                @pl.when(l >= 2)
                def _():
                    wait_write(l - 2, b)

                transpose_scale(l, s, b)

                @pl.when(i < _L // 4 - 1)
                def _():
                    fill_idx(l + 4, s)
                    start_gather(s)

                start_write(l, b)
            return 0

        lax.fori_loop(0, _L // 4, body, 0)

        for b in range(2):
            wait_write(_L - 2 + b, b)

    return sc_embed


_sc_embed = _make_sc_kernel()


def kernel(tokens, table):
    tok_t = tokens.T
    ltp = jnp.pad(table, ((0, 0), (0, 64)))
    out = _sc_embed(tok_t, ltp)
    return out.transpose(2, 0, 1)
